# Initial kernel scaffold; baseline (speedup 1.0000x reference)
import jax
import jax.numpy as jnp
from jax.experimental import pallas as pl


def kernel(position_ids, PosEnc):
    raise NotImplementedError("write your pallas kernel here")


# speedup vs baseline: 1.3408x; 1.3408x over previous
import functools

import jax
import jax.numpy as jnp
from jax import lax
from jax.experimental import pallas as pl
from jax.experimental.pallas import tpu as pltpu
from jax.experimental.pallas import tpu_sc as plsc

_NC = 2
_NS = 16
_NW = _NC * _NS

_D = 1024
_C = 32
_NBUF = 3


def _make_gather(B):
    assert B % _NW == 0
    bpw = B // _NW
    assert bpw % _C == 0
    nch = bpw // _C

    mesh = plsc.VectorSubcoreMesh(core_axis_name="c", subcore_axis_name="s")

    @functools.partial(
        pl.kernel,
        mesh=mesh,
        out_type=jax.ShapeDtypeStruct((B, _D), jnp.float32),
        scratch_types=[
            pltpu.VMEM((nch, _C), jnp.int32),
            [pltpu.VMEM((_C, _D), jnp.float32) for _ in range(_NBUF)],
            [pltpu.SemaphoreType.DMA for _ in range(_NBUF)],
            [pltpu.SemaphoreType.DMA for _ in range(_NBUF)],
        ],
    )
    def gather(idx_hbm, table_hbm, out_hbm, idx_v, bufs, gsems, wsems):
        wid = lax.axis_index("s") * _NC + lax.axis_index("c")
        base = wid * bpw
        pltpu.sync_copy(idx_hbm.at[wid], idx_v)

        gathers = [None] * nch
        writes = [None] * nch

        def start_gather(j):
            b = j % _NBUF
            cp = pltpu.make_async_copy(
                table_hbm.at[idx_v.at[j]], bufs[b], gsems[b])
            cp.start()
            gathers[j] = cp

        def start_write(j):
            b = j % _NBUF
            cp = pltpu.make_async_copy(
                bufs[b], out_hbm.at[pl.ds(base + j * _C, _C)], wsems[b])
            cp.start()
            writes[j] = cp

        for j in range(min(_NBUF, nch)):
            start_gather(j)
        for j in range(nch):
            gathers[j].wait()
            start_write(j)
            nxt = j + _NBUF
            if nxt < nch:
                writes[j].wait()
                start_gather(nxt)
        for j in range(max(0, nch - _NBUF), nch):
            writes[j].wait()

    return gather


def kernel(position_ids, PosEnc):
    B = position_ids.shape[0]
    table = PosEnc.reshape(PosEnc.shape[1], _D)
    idx = position_ids.astype(jnp.int32).reshape(_NW, B // (_NW * _C), _C)
    out = _make_gather(B)(idx, table)
    return out.reshape(1, B, _D)


# speedup vs baseline: 2.1887x; 1.6324x over previous
import functools

import jax
import jax.numpy as jnp
from jax import lax
from jax.experimental import pallas as pl
from jax.experimental.pallas import tpu as pltpu
from jax.experimental.pallas import tpu_sc as plsc

_NC = 2
_NS = 16
_NW = _NC * _NS

_D = 1024
_CT = _D // 128
_SUB = 16
_NBUF = 6
_L = 16


def _make_gather(B):
    assert B % (_NW * _SUB) == 0
    bpw = B // _NW
    steps = bpw // _SUB
    g_rows = _SUB * _CT
    gpw = bpw * _CT

    mesh = plsc.VectorSubcoreMesh(core_axis_name="c", subcore_axis_name="s")

    @functools.partial(
        pl.kernel,
        mesh=mesh,
        compiler_params=pltpu.CompilerParams(
            use_tc_tiling_on_sc=False, needs_layout_passes=False),
        out_type=jax.ShapeDtypeStruct((B * _CT, 128), jnp.float32),
        scratch_types=[
            pltpu.VMEM((bpw,), jnp.int32),
            pltpu.VMEM((gpw,), jnp.int32),
            [pltpu.VMEM((g_rows, 128), jnp.float32) for _ in range(_NBUF)],
            [pltpu.SemaphoreType.DMA for _ in range(_NBUF)],
            [pltpu.SemaphoreType.DMA for _ in range(_NBUF)],
        ],
    )
    def gather(idx_hbm, table_hbm, out_hbm, idx_v, j_v, bufs, gsems, wsems):
        wid = lax.axis_index("s") * _NC + lax.axis_index("c")
        pltpu.sync_copy(idx_hbm.at[wid], idx_v)

        lanes = lax.iota(jnp.int32, _L)

        def build_j(s):
            for k in range(s * (g_rows // _L), (s + 1) * (g_rows // _L)):
                g = k * _L + lanes
                row = ((g >> 6) << 3) + (g & 7)
                ct = (g >> 3) & 7
                tab = plsc.load_gather(idx_v, [row])
                j_v[pl.ds(k * _L, _L)] = (tab << 3) | ct

        obase = wid * gpw
        gathers = [None] * steps
        writes = [None] * steps

        def start_gather(s):
            b = s % _NBUF
            cp = pltpu.make_async_copy(
                table_hbm.at[j_v.at[pl.ds(s * g_rows, g_rows)]],
                bufs[b], gsems[b])
            cp.start()
            gathers[s] = cp

        def start_write(s):
            b = s % _NBUF
            cp = pltpu.make_async_copy(
                bufs[b], out_hbm.at[pl.ds(obase + s * g_rows, g_rows)],
                wsems[b])
            cp.start()
            writes[s] = cp

        for s in range(min(_NBUF, steps)):
            build_j(s)
            start_gather(s)
        for s in range(steps):
            nxt = s + _NBUF
            if nxt < steps:
                build_j(nxt)
            gathers[s].wait()
            start_write(s)
            if nxt < steps:
                writes[s].wait()
                start_gather(nxt)
        for s in range(max(0, steps - _NBUF), steps):
            writes[s].wait()

    return gather


def kernel(position_ids, PosEnc):
    B = position_ids.shape[0]
    table2 = PosEnc.reshape(PosEnc.shape[1] * _CT, 128)
    idx = position_ids.astype(jnp.int32).reshape(_NW, B // _NW)
    out = _make_gather(B)(idx, table2)
    out4 = out.reshape(B // 8, 8, 8, 128)
    return out4.transpose(0, 2, 1, 3).reshape(1, B, _D)
